# baseline (speedup 1.0000x reference)
import functools

import jax
import jax.numpy as jnp
from jax import lax
from jax.experimental import pallas as pl
from jax.experimental.pallas import tpu as pltpu
from jax.experimental.pallas import tpu_sc as plsc

N = 10000
E = 320000
D = 128
NC, NS, L = 2, 16, 16
NW = NC * NS
EPW = E // NW
RPT = 320
NPAD = NW * RPT
GCHUNK = 400
GSUB = 80
SCHUNK = 3200
RING = 4096
RBATCH = 256

_NEG = float("-inf")


def _pre_body(x_ref, w1_ref, b1_ref, a_ref, b_ref):
    xb = x_ref[...]
    w1a = w1_ref[:D, :]
    w1b = w1_ref[D:, :]
    a_ref[...] = jnp.dot(xb, w1a - w1b, preferred_element_type=jnp.float32) + b1_ref[...]
    b_ref[...] = jnp.dot(xb, w1b, preferred_element_type=jnp.float32)


def _pre(x, W1, b1):
    grid = 10
    blk = N // grid
    return pl.pallas_call(
        _pre_body,
        grid=(grid,),
        in_specs=[
            pl.BlockSpec((blk, D), lambda i: (i, 0)),
            pl.BlockSpec((2 * D, D), lambda i: (0, 0)),
            pl.BlockSpec((1, D), lambda i: (0, 0)),
        ],
        out_specs=[
            pl.BlockSpec((blk, D), lambda i: (i, 0)),
            pl.BlockSpec((blk, D), lambda i: (i, 0)),
        ],
        out_shape=[
            jax.ShapeDtypeStruct((N, D), jnp.float32),
            jax.ShapeDtypeStruct((N, D), jnp.float32),
        ],
    )(x, W1, b1)


def _gather_body(a_hbm, b_hbm, dst_hbm, src_hbm, g1_hbm, g2_hbm,
                 idxd_v, idxs_v, bufa_v, bufb_v, sem_a, sem_b):
    wid = lax.axis_index("s") * NC + lax.axis_index("c")
    ebase = wid * EPW

    def chunk(ci, carry):
        cbase = ebase + ci * GCHUNK
        pltpu.sync_copy(dst_hbm.at[pl.ds(cbase, GCHUNK)], idxd_v)
        pltpu.sync_copy(src_hbm.at[pl.ds(cbase, GCHUNK)], idxs_v)
        das = []
        dbs = []
        for k in range(GCHUNK // GSUB):
            sl = pl.ds(k * GSUB, GSUB)
            das.append(pltpu.async_copy(a_hbm.at[idxd_v.at[sl]], bufa_v.at[sl], sem_a))
            dbs.append(pltpu.async_copy(b_hbm.at[idxs_v.at[sl]], bufb_v.at[sl], sem_b))
        for d in das:
            d.wait()
        for d in dbs:
            d.wait()
        pltpu.sync_copy(bufa_v, g1_hbm.at[pl.ds(cbase, GCHUNK)])
        pltpu.sync_copy(bufb_v, g2_hbm.at[pl.ds(cbase, GCHUNK)])
        return carry

    lax.fori_loop(0, EPW // GCHUNK, chunk, 0)


def _gather(A, B, dst, src):
    f = pl.kernel(
        _gather_body,
        out_type=[
            jax.ShapeDtypeStruct((E, D), jnp.float32),
            jax.ShapeDtypeStruct((E, D), jnp.float32),
        ],
        mesh=plsc.VectorSubcoreMesh(core_axis_name="c", subcore_axis_name="s"),
        compiler_params=pltpu.CompilerParams(needs_layout_passes=False),
        scratch_types=[
            pltpu.VMEM((GCHUNK,), jnp.int32),
            pltpu.VMEM((GCHUNK,), jnp.int32),
            pltpu.VMEM((GCHUNK, D), jnp.float32),
            pltpu.VMEM((GCHUNK, D), jnp.float32),
            pltpu.SemaphoreType.DMA,
            pltpu.SemaphoreType.DMA,
        ],
    )
    return f(A, B, dst, src)


def _mlp_body(g1_ref, g2_ref, w2_ref, b2_ref, h2_ref):
    h1 = g1_ref[...] + g2_ref[...]
    m = h1 * jnp.tanh(jax.nn.softplus(h1))
    h2_ref[...] = jnp.dot(m, w2_ref[...], preferred_element_type=jnp.float32) + b2_ref[...]


def _mlp(G1, G2, W2, b2):
    blk = 512
    grid = E // blk
    return pl.pallas_call(
        _mlp_body,
        grid=(grid,),
        in_specs=[
            pl.BlockSpec((blk, D), lambda i: (i, 0)),
            pl.BlockSpec((blk, D), lambda i: (i, 0)),
            pl.BlockSpec((D, D), lambda i: (0, 0)),
            pl.BlockSpec((1, D), lambda i: (0, 0)),
        ],
        out_specs=pl.BlockSpec((blk, D), lambda i: (i, 0)),
        out_shape=jax.ShapeDtypeStruct((E, D), jnp.float32),
    )(G1, G2, W2, b2)


def _segmax_body(h2_hbm, dst_hbm, agg_hbm,
                 dstb_v, mloc_v, mid_v, rows_v, agg_v, sem):
    wid = lax.axis_index("s") * NC + lax.axis_index("c")
    lo = wid * RPT
    hi = lo + RPT
    neg = jnp.full((L,), _NEG, dtype=jnp.float32)
    iota = lax.iota(jnp.int32, L)
    trash = jnp.full((L,), RPT, jnp.int32)

    def init(i, carry):
        for cc in range(D // L):
            agg_v[i, pl.ds(cc * L, L)] = neg
        return carry

    lax.fori_loop(0, RPT + 1, init, 0)

    def do_rmw(roff):
        def rmw(i, c2):
            r = mloc_v[pl.ds(roff + i, L)][0]
            for cc in range(D // L):
                sl = pl.ds(cc * L, L)
                agg_v[r, sl] = jnp.maximum(agg_v[r, sl], rows_v[i, sl])
            return c2

        lax.fori_loop(0, RBATCH, rmw, 0)

    def chunk(ci, carry):
        cur0, fl0 = carry
        cbase = ci * SCHUNK
        pltpu.sync_copy(dst_hbm.at[pl.ds(cbase, SCHUNK)], dstb_v)

        def scan32(g, cur):
            d16a = dstb_v[pl.ds(g * 2 * L, L)]
            d16b = dstb_v[pl.ds(g * 2 * L + L, L)]
            ma = (d16a >= lo) & (d16a < hi)
            mb = (d16b >= lo) & (d16b < hi)
            pca = plsc.cumsum(jnp.where(ma, jnp.int32(1), jnp.int32(0)))
            pcb = plsc.cumsum(jnp.where(mb, jnp.int32(1), jnp.int32(0)))
            ca = pca[15]
            posa = jnp.where(ma, (cur + pca - 1) & (RING - 1), RING + iota)
            posb = jnp.where(mb, (cur + ca + pcb - 1) & (RING - 1), RING + iota)
            plsc.store_scatter(mloc_v, [posa], d16a - lo)
            plsc.store_scatter(mid_v, [posa], cbase + g * 2 * L + iota)
            plsc.store_scatter(mloc_v, [posb], d16b - lo)
            plsc.store_scatter(mid_v, [posb], cbase + g * 2 * L + L + iota)
            return cur + ca + pcb[15]

        cur1 = lax.fori_loop(0, SCHUNK // (2 * L), scan32, cur0)

        def wcond(st):
            c2, f2 = st
            return c2 - f2 >= RBATCH

        def wbody(st):
            c2, f2 = st
            roff = pl.multiple_of(f2 & (RING - 1), RBATCH)
            d1 = pltpu.async_copy(h2_hbm.at[mid_v.at[pl.ds(roff, 128)]],
                                  rows_v.at[pl.ds(0, 128)], sem)
            d2 = pltpu.async_copy(h2_hbm.at[mid_v.at[pl.ds(roff + 128, 128)]],
                                  rows_v.at[pl.ds(128, 128)], sem)
            d1.wait()
            d2.wait()
            do_rmw(roff)
            return (c2, f2 + RBATCH)

        return lax.while_loop(wcond, wbody, (cur1, fl0))

    cur, fl = lax.fori_loop(0, E // SCHUNK, chunk,
                            (jnp.int32(0), jnp.int32(0)))

    pad_base = wid * RBATCH
    for j in range(RBATCH // L):
        pos = (cur + j * L + iota) & (RING - 1)
        plsc.store_scatter(mid_v, [pos], pad_base + j * L + iota)
        plsc.store_scatter(mloc_v, [pos], trash)

    def dcond(st):
        c2, f2 = st
        return f2 < c2

    def dbody(st):
        c2, f2 = st
        roff = pl.multiple_of(f2 & (RING - 1), RBATCH)
        d1 = pltpu.async_copy(h2_hbm.at[mid_v.at[pl.ds(roff, 128)]],
                              rows_v.at[pl.ds(0, 128)], sem)
        d2 = pltpu.async_copy(h2_hbm.at[mid_v.at[pl.ds(roff + 128, 128)]],
                              rows_v.at[pl.ds(128, 128)], sem)
        d1.wait()
        d2.wait()
        do_rmw(roff)
        return (c2, f2 + RBATCH)

    lax.while_loop(dcond, dbody, (cur, fl))
    pltpu.sync_copy(agg_v.at[pl.ds(0, RPT)], agg_hbm.at[pl.ds(lo, RPT)])


def _segmax(H2, dst):
    f = pl.kernel(
        _segmax_body,
        out_type=jax.ShapeDtypeStruct((NPAD, D), jnp.float32),
        mesh=plsc.VectorSubcoreMesh(core_axis_name="c", subcore_axis_name="s"),
        compiler_params=pltpu.CompilerParams(needs_layout_passes=False),
        scratch_types=[
            pltpu.VMEM((SCHUNK,), jnp.int32),
            pltpu.VMEM((RING + L,), jnp.int32),
            pltpu.VMEM((RING + L,), jnp.int32),
            pltpu.VMEM((RBATCH, D), jnp.float32),
            pltpu.VMEM((RPT + 1, D), jnp.float32),
            pltpu.SemaphoreType.DMA,
        ],
    )
    return f(H2, dst)


def _bn_body(agg_ref, gamma_ref, beta_ref, y_ref):
    a = agg_ref[...]
    a = jnp.where(a == _NEG, 0.0, a)
    mean = jnp.mean(a, axis=0, keepdims=True)
    var = jnp.mean((a - mean) ** 2, axis=0, keepdims=True)
    y_ref[...] = gamma_ref[...] * (a - mean) / jnp.sqrt(var + 1e-5) + beta_ref[...]


def _bn(agg, gamma, beta):
    return pl.pallas_call(
        _bn_body,
        in_specs=[
            pl.BlockSpec((N, D), lambda: (0, 0)),
            pl.BlockSpec((1, D), lambda: (0, 0)),
            pl.BlockSpec((1, D), lambda: (0, 0)),
        ],
        out_specs=pl.BlockSpec((N, D), lambda: (0, 0)),
        out_shape=jax.ShapeDtypeStruct((N, D), jnp.float32),
    )(agg, gamma, beta)


def kernel(x, edge_index, edge_attr, W1, b1, W2, b2, gamma, beta):
    src = edge_index[0]
    dst = edge_index[1]
    A, B = _pre(x, W1, b1.reshape(1, D))
    G1, G2 = _gather(A, B, dst, src)
    H2 = _mlp(G1, G2, W2, b2.reshape(1, D))
    aggp = _segmax(H2, dst)
    y = _bn(aggp[:N], gamma.reshape(1, D), beta.reshape(1, D))
    return (y, edge_index, edge_attr)


# speedup vs baseline: 1.1974x; 1.1974x over previous
import functools

import jax
import jax.numpy as jnp
from jax import lax
from jax.experimental import pallas as pl
from jax.experimental.pallas import tpu as pltpu
from jax.experimental.pallas import tpu_sc as plsc

N = 10000
E = 320000
D = 128
NC, NS, L = 2, 16, 16
NW = NC * NS
EPW = E // NW
RPT = 320
NPAD = NW * RPT
GCHUNK = 400
GSUB = 80
SCHUNK = 3200
RING = 4096
RBATCH = 256
CAP = E + RBATCH

_NEG = float("-inf")


def _pre_body(x_ref, w1_ref, b1_ref, a_ref, b_ref):
    xb = x_ref[...]
    w1a = w1_ref[:D, :]
    w1b = w1_ref[D:, :]
    a_ref[...] = jnp.dot(xb, w1a - w1b, preferred_element_type=jnp.float32) + b1_ref[...]
    b_ref[...] = jnp.dot(xb, w1b, preferred_element_type=jnp.float32)


def _pre(x, W1, b1):
    grid = 10
    blk = N // grid
    return pl.pallas_call(
        _pre_body,
        grid=(grid,),
        in_specs=[
            pl.BlockSpec((blk, D), lambda i: (i, 0)),
            pl.BlockSpec((2 * D, D), lambda i: (0, 0)),
            pl.BlockSpec((1, D), lambda i: (0, 0)),
        ],
        out_specs=[
            pl.BlockSpec((blk, D), lambda i: (i, 0)),
            pl.BlockSpec((blk, D), lambda i: (i, 0)),
        ],
        out_shape=[
            jax.ShapeDtypeStruct((N, D), jnp.float32),
            jax.ShapeDtypeStruct((N, D), jnp.float32),
        ],
    )(x, W1, b1)


def _gather_body(a_hbm, b_hbm, dst_hbm, src_hbm, g1_hbm, g2_hbm,
                 idxd_v, idxs_v, bufa_v, bufb_v, sem_a, sem_b):
    wid = lax.axis_index("s") * NC + lax.axis_index("c")
    ebase = wid * EPW

    def chunk(ci, carry):
        cbase = ebase + ci * GCHUNK
        pltpu.sync_copy(dst_hbm.at[pl.ds(cbase, GCHUNK)], idxd_v)
        pltpu.sync_copy(src_hbm.at[pl.ds(cbase, GCHUNK)], idxs_v)
        das = []
        dbs = []
        for k in range(GCHUNK // GSUB):
            sl = pl.ds(k * GSUB, GSUB)
            das.append(pltpu.async_copy(a_hbm.at[idxd_v.at[sl]], bufa_v.at[sl], sem_a))
            dbs.append(pltpu.async_copy(b_hbm.at[idxs_v.at[sl]], bufb_v.at[sl], sem_b))
        for d in das:
            d.wait()
        for d in dbs:
            d.wait()
        pltpu.sync_copy(bufa_v, g1_hbm.at[pl.ds(cbase, GCHUNK)])
        pltpu.sync_copy(bufb_v, g2_hbm.at[pl.ds(cbase, GCHUNK)])
        return carry

    lax.fori_loop(0, EPW // GCHUNK, chunk, 0)


def _gather(A, B, dst, src):
    f = pl.kernel(
        _gather_body,
        out_type=[
            jax.ShapeDtypeStruct((E, D), jnp.float32),
            jax.ShapeDtypeStruct((E, D), jnp.float32),
        ],
        mesh=plsc.VectorSubcoreMesh(core_axis_name="c", subcore_axis_name="s"),
        compiler_params=pltpu.CompilerParams(needs_layout_passes=False),
        scratch_types=[
            pltpu.VMEM((GCHUNK,), jnp.int32),
            pltpu.VMEM((GCHUNK,), jnp.int32),
            pltpu.VMEM((GCHUNK, D), jnp.float32),
            pltpu.VMEM((GCHUNK, D), jnp.float32),
            pltpu.SemaphoreType.DMA,
            pltpu.SemaphoreType.DMA,
        ],
    )
    return f(A, B, dst, src)


def _mlp_body(g1_ref, g2_ref, w2_ref, b2_ref, h2_ref):
    h1 = g1_ref[...] + g2_ref[...]
    m = h1 * jnp.tanh(jax.nn.softplus(h1))
    h2_ref[...] = jnp.dot(m, w2_ref[...], preferred_element_type=jnp.float32) + b2_ref[...]


def _mlp(G1, G2, W2, b2):
    blk = 512
    grid = E // blk
    return pl.pallas_call(
        _mlp_body,
        grid=(grid,),
        in_specs=[
            pl.BlockSpec((blk, D), lambda i: (i, 0)),
            pl.BlockSpec((blk, D), lambda i: (i, 0)),
            pl.BlockSpec((D, D), lambda i: (0, 0)),
            pl.BlockSpec((1, D), lambda i: (0, 0)),
        ],
        out_specs=pl.BlockSpec((blk, D), lambda i: (i, 0)),
        out_shape=jax.ShapeDtypeStruct((E, D), jnp.float32),
    )(G1, G2, W2, b2)


def _scan_body(dst_hbm, loc_hbm, eid_hbm, cnt_hbm, dstb_v, mloc_v, mid_v, cnt_v):
    wid = lax.axis_index("s") * NC + lax.axis_index("c")
    lo = wid * RPT
    hi = lo + RPT
    iota = lax.iota(jnp.int32, L)
    trash = jnp.full((L,), RPT, jnp.int32)

    def dump(dmp):
        roff = pl.multiple_of(dmp & (RING - 1), 256)
        hoff = pl.multiple_of(wid * CAP + dmp, 256)
        pltpu.sync_copy(mloc_v.at[pl.ds(roff, 256)], loc_hbm.at[pl.ds(hoff, 256)])
        pltpu.sync_copy(mid_v.at[pl.ds(roff, 256)], eid_hbm.at[pl.ds(hoff, 256)])

    def chunk(ci, carry):
        cur0, dmp0 = carry
        cbase = ci * SCHUNK
        pltpu.sync_copy(dst_hbm.at[pl.ds(cbase, SCHUNK)], dstb_v)

        def scan32(g, cur):
            d16a = dstb_v[pl.ds(g * 2 * L, L)]
            d16b = dstb_v[pl.ds(g * 2 * L + L, L)]
            ma = (d16a >= lo) & (d16a < hi)
            mb = (d16b >= lo) & (d16b < hi)
            pca = plsc.cumsum(jnp.where(ma, jnp.int32(1), jnp.int32(0)))
            pcb = plsc.cumsum(jnp.where(mb, jnp.int32(1), jnp.int32(0)))
            ca = pca[15]
            posa = jnp.where(ma, (cur + pca - 1) & (RING - 1), RING + iota)
            posb = jnp.where(mb, (cur + ca + pcb - 1) & (RING - 1), RING + iota)
            plsc.store_scatter(mloc_v, [posa], d16a - lo)
            plsc.store_scatter(mid_v, [posa], cbase + g * 2 * L + iota)
            plsc.store_scatter(mloc_v, [posb], d16b - lo)
            plsc.store_scatter(mid_v, [posb], cbase + g * 2 * L + L + iota)
            return cur + ca + pcb[15]

        cur1 = lax.fori_loop(0, SCHUNK // (2 * L), scan32, cur0)

        def wcond(st):
            c2, f2 = st
            return c2 - f2 >= 256

        def wbody(st):
            c2, f2 = st
            dump(f2)
            return (c2, f2 + 256)

        return lax.while_loop(wcond, wbody, (cur1, dmp0))

    cur, dmp = lax.fori_loop(0, E // SCHUNK, chunk,
                             (jnp.int32(0), jnp.int32(0)))

    pad_base = wid * RBATCH
    for j in range(RBATCH // L):
        pos = (cur + j * L + iota) & (RING - 1)
        plsc.store_scatter(mid_v, [pos], pad_base + j * L + iota)
        plsc.store_scatter(mloc_v, [pos], trash)

    def dcond(st):
        c2, f2 = st
        return f2 < c2

    def dbody(st):
        c2, f2 = st
        dump(f2)
        return (c2, f2 + 256)

    lax.while_loop(dcond, dbody, (cur, dmp))
    cnt_v[pl.ds(0, L)] = jnp.full((L,), 0, jnp.int32) + cur
    pltpu.sync_copy(cnt_v, cnt_hbm.at[pl.ds(pl.multiple_of(wid * L, L), L)])


def _scan(dst):
    f = pl.kernel(
        _scan_body,
        out_type=[
            jax.ShapeDtypeStruct((NW * CAP,), jnp.int32),
            jax.ShapeDtypeStruct((NW * CAP,), jnp.int32),
            jax.ShapeDtypeStruct((NW * L,), jnp.int32),
        ],
        mesh=plsc.VectorSubcoreMesh(core_axis_name="c", subcore_axis_name="s"),
        compiler_params=pltpu.CompilerParams(needs_layout_passes=False),
        scratch_types=[
            pltpu.VMEM((SCHUNK,), jnp.int32),
            pltpu.VMEM((RING + L,), jnp.int32),
            pltpu.VMEM((RING + L,), jnp.int32),
            pltpu.VMEM((L,), jnp.int32),
        ],
    )
    return f(dst)


def _flush_body(h2_hbm, loc_hbm, eid_hbm, cnt_hbm, agg_hbm,
                lbuf_v, ebuf_v, cnt_v, rows_v, agg_v, sem):
    wid = lax.axis_index("s") * NC + lax.axis_index("c")
    lo = wid * RPT
    neg = jnp.full((L,), _NEG, dtype=jnp.float32)

    def init(i, carry):
        for cc in range(D // L):
            agg_v[i, pl.ds(cc * L, L)] = neg
        return carry

    lax.fori_loop(0, RPT + 1, init, 0)

    pltpu.sync_copy(cnt_hbm.at[pl.ds(pl.multiple_of(wid * L, L), L)], cnt_v)
    cnt = cnt_v[pl.ds(0, L)][0]
    nb = (cnt + RBATCH - 1) // RBATCH

    def batch(b, carry):
        boff = pl.multiple_of(b * RBATCH, RBATCH)
        hoff = pl.multiple_of(wid * CAP + boff, RBATCH)
        pltpu.sync_copy(loc_hbm.at[pl.ds(hoff, RBATCH)], lbuf_v.at[pl.ds(0, RBATCH)])
        pltpu.sync_copy(eid_hbm.at[pl.ds(hoff, RBATCH)], ebuf_v)
        d1 = pltpu.async_copy(h2_hbm.at[ebuf_v.at[pl.ds(0, 128)]],
                              rows_v.at[pl.ds(0, 128)], sem)
        d2 = pltpu.async_copy(h2_hbm.at[ebuf_v.at[pl.ds(128, 128)]],
                              rows_v.at[pl.ds(128, 128)], sem)
        d1.wait()
        d2.wait()

        def rmw(i, c2):
            r = lbuf_v[pl.ds(i, L)][0]
            for cc in range(D // L):
                sl = pl.ds(cc * L, L)
                agg_v[r, sl] = jnp.maximum(agg_v[r, sl], rows_v[i, sl])
            return c2

        lax.fori_loop(0, RBATCH, rmw, 0)
        return carry

    lax.fori_loop(0, nb, batch, 0)
    pltpu.sync_copy(agg_v.at[pl.ds(0, RPT)], agg_hbm.at[pl.ds(lo, RPT)])


def _flush(H2, LOC, EID, CNT):
    f = pl.kernel(
        _flush_body,
        out_type=jax.ShapeDtypeStruct((NPAD, D), jnp.float32),
        mesh=plsc.VectorSubcoreMesh(core_axis_name="c", subcore_axis_name="s"),
        compiler_params=pltpu.CompilerParams(needs_layout_passes=False),
        scratch_types=[
            pltpu.VMEM((RBATCH + L,), jnp.int32),
            pltpu.VMEM((RBATCH,), jnp.int32),
            pltpu.VMEM((L,), jnp.int32),
            pltpu.VMEM((RBATCH, D), jnp.float32),
            pltpu.VMEM((RPT + 1, D), jnp.float32),
            pltpu.SemaphoreType.DMA,
        ],
    )
    return f(H2, LOC, EID, CNT)


def _bn_body(agg_ref, gamma_ref, beta_ref, y_ref):
    a = agg_ref[...]
    a = jnp.where(a == _NEG, 0.0, a)
    mean = jnp.mean(a, axis=0, keepdims=True)
    var = jnp.mean((a - mean) ** 2, axis=0, keepdims=True)
    y_ref[...] = gamma_ref[...] * (a - mean) / jnp.sqrt(var + 1e-5) + beta_ref[...]


def _bn(agg, gamma, beta):
    return pl.pallas_call(
        _bn_body,
        in_specs=[
            pl.BlockSpec((N, D), lambda: (0, 0)),
            pl.BlockSpec((1, D), lambda: (0, 0)),
            pl.BlockSpec((1, D), lambda: (0, 0)),
        ],
        out_specs=pl.BlockSpec((N, D), lambda: (0, 0)),
        out_shape=jax.ShapeDtypeStruct((N, D), jnp.float32),
    )(agg, gamma, beta)


def kernel(x, edge_index, edge_attr, W1, b1, W2, b2, gamma, beta):
    src = edge_index[0]
    dst = edge_index[1]
    A, B = _pre(x, W1, b1.reshape(1, D))
    G1, G2 = _gather(A, B, dst, src)
    H2 = _mlp(G1, G2, W2, b2.reshape(1, D))
    LOC, EID, CNT = _scan(dst)
    aggp = _flush(H2, LOC, EID, CNT)
    y = _bn(aggp[:N], gamma.reshape(1, D), beta.reshape(1, D))
    return (y, edge_index, edge_attr)


# speedup vs baseline: 1.2239x; 1.0221x over previous
import functools

import jax
import jax.numpy as jnp
from jax import lax
from jax.experimental import pallas as pl
from jax.experimental.pallas import tpu as pltpu
from jax.experimental.pallas import tpu_sc as plsc

N = 10000
E = 320000
D = 128
NC, NS, L = 2, 16, 16
NW = NC * NS
EPW = E // NW
RPT = 320
NPAD = NW * RPT
GCHUNK = 400
GSUB = 80
SCHUNK = 3200
RING = 4096
RBATCH = 256
CAP = E + RBATCH

_NEG = float("-inf")


def _pre_body(x_ref, w1_ref, b1_ref, a_ref, b_ref):
    xb = x_ref[...]
    w1a = w1_ref[:D, :]
    w1b = w1_ref[D:, :]
    a_ref[...] = jnp.dot(xb, w1a - w1b, preferred_element_type=jnp.float32) + b1_ref[...]
    b_ref[...] = jnp.dot(xb, w1b, preferred_element_type=jnp.float32)


def _pre(x, W1, b1):
    grid = 10
    blk = N // grid
    return pl.pallas_call(
        _pre_body,
        grid=(grid,),
        in_specs=[
            pl.BlockSpec((blk, D), lambda i: (i, 0)),
            pl.BlockSpec((2 * D, D), lambda i: (0, 0)),
            pl.BlockSpec((1, D), lambda i: (0, 0)),
        ],
        out_specs=[
            pl.BlockSpec((blk, D), lambda i: (i, 0)),
            pl.BlockSpec((blk, D), lambda i: (i, 0)),
        ],
        out_shape=[
            jax.ShapeDtypeStruct((N, D), jnp.float32),
            jax.ShapeDtypeStruct((N, D), jnp.float32),
        ],
    )(x, W1, b1)


def _gather_body(a_hbm, b_hbm, dst_hbm, src_hbm, g1_hbm, g2_hbm,
                 idxd_v, idxs_v, bufa_v, bufb_v, sem_a, sem_b):
    wid = lax.axis_index("s") * NC + lax.axis_index("c")
    ebase = wid * EPW

    def chunk(ci, carry):
        cbase = ebase + ci * GCHUNK
        pltpu.sync_copy(dst_hbm.at[pl.ds(cbase, GCHUNK)], idxd_v)
        pltpu.sync_copy(src_hbm.at[pl.ds(cbase, GCHUNK)], idxs_v)
        das = []
        dbs = []
        for k in range(GCHUNK // GSUB):
            sl = pl.ds(k * GSUB, GSUB)
            das.append(pltpu.async_copy(a_hbm.at[idxd_v.at[sl]], bufa_v.at[sl], sem_a))
            dbs.append(pltpu.async_copy(b_hbm.at[idxs_v.at[sl]], bufb_v.at[sl], sem_b))
        for d in das:
            d.wait()
        for d in dbs:
            d.wait()
        pltpu.sync_copy(bufa_v, g1_hbm.at[pl.ds(cbase, GCHUNK)])
        pltpu.sync_copy(bufb_v, g2_hbm.at[pl.ds(cbase, GCHUNK)])
        return carry

    lax.fori_loop(0, EPW // GCHUNK, chunk, 0)


def _gather(A, B, dst, src):
    f = pl.kernel(
        _gather_body,
        out_type=[
            jax.ShapeDtypeStruct((E, D), jnp.float32),
            jax.ShapeDtypeStruct((E, D), jnp.float32),
        ],
        mesh=plsc.VectorSubcoreMesh(core_axis_name="c", subcore_axis_name="s"),
        compiler_params=pltpu.CompilerParams(needs_layout_passes=False),
        scratch_types=[
            pltpu.VMEM((GCHUNK,), jnp.int32),
            pltpu.VMEM((GCHUNK,), jnp.int32),
            pltpu.VMEM((GCHUNK, D), jnp.float32),
            pltpu.VMEM((GCHUNK, D), jnp.float32),
            pltpu.SemaphoreType.DMA,
            pltpu.SemaphoreType.DMA,
        ],
    )
    return f(A, B, dst, src)


def _mlp_body(g1_ref, g2_ref, w2_ref, b2_ref, h2_ref):
    h1 = g1_ref[...] + g2_ref[...]
    m = h1 * jnp.tanh(jax.nn.softplus(h1))
    h2_ref[...] = jnp.dot(m, w2_ref[...], preferred_element_type=jnp.float32) + b2_ref[...]


def _mlp(G1, G2, W2, b2):
    blk = 512
    grid = E // blk
    return pl.pallas_call(
        _mlp_body,
        grid=(grid,),
        in_specs=[
            pl.BlockSpec((blk, D), lambda i: (i, 0)),
            pl.BlockSpec((blk, D), lambda i: (i, 0)),
            pl.BlockSpec((D, D), lambda i: (0, 0)),
            pl.BlockSpec((1, D), lambda i: (0, 0)),
        ],
        out_specs=pl.BlockSpec((blk, D), lambda i: (i, 0)),
        out_shape=jax.ShapeDtypeStruct((E, D), jnp.float32),
    )(G1, G2, W2, b2)


def _scan_body(dst_hbm, loc_hbm, eid_hbm, cnt_hbm, dstb_v, mloc_v, mid_v, cnt_v):
    wid = lax.axis_index("s") * NC + lax.axis_index("c")
    lo = wid * RPT
    hi = lo + RPT
    iota = lax.iota(jnp.int32, L)
    trash = jnp.full((L,), RPT, jnp.int32)

    def dump(dmp):
        roff = pl.multiple_of(dmp & (RING - 1), 256)
        hoff = pl.multiple_of(wid * CAP + dmp, 256)
        pltpu.sync_copy(mloc_v.at[pl.ds(roff, 256)], loc_hbm.at[pl.ds(hoff, 256)])
        pltpu.sync_copy(mid_v.at[pl.ds(roff, 256)], eid_hbm.at[pl.ds(hoff, 256)])

    def chunk(ci, carry):
        cur0, dmp0 = carry
        cbase = ci * SCHUNK
        pltpu.sync_copy(dst_hbm.at[pl.ds(cbase, SCHUNK)], dstb_v)

        def scan32(g, cur):
            d16a = dstb_v[pl.ds(g * 2 * L, L)]
            d16b = dstb_v[pl.ds(g * 2 * L + L, L)]
            ma = (d16a >= lo) & (d16a < hi)
            mb = (d16b >= lo) & (d16b < hi)
            pca = plsc.cumsum(jnp.where(ma, jnp.int32(1), jnp.int32(0)))
            pcb = plsc.cumsum(jnp.where(mb, jnp.int32(1), jnp.int32(0)))
            ca = pca[15]
            posa = jnp.where(ma, (cur + pca - 1) & (RING - 1), RING + iota)
            posb = jnp.where(mb, (cur + ca + pcb - 1) & (RING - 1), RING + iota)
            plsc.store_scatter(mloc_v, [posa], d16a - lo)
            plsc.store_scatter(mid_v, [posa], cbase + g * 2 * L + iota)
            plsc.store_scatter(mloc_v, [posb], d16b - lo)
            plsc.store_scatter(mid_v, [posb], cbase + g * 2 * L + L + iota)
            return cur + ca + pcb[15]

        cur1 = lax.fori_loop(0, SCHUNK // (2 * L), scan32, cur0)

        def wcond(st):
            c2, f2 = st
            return c2 - f2 >= 256

        def wbody(st):
            c2, f2 = st
            dump(f2)
            return (c2, f2 + 256)

        return lax.while_loop(wcond, wbody, (cur1, dmp0))

    cur, dmp = lax.fori_loop(0, E // SCHUNK, chunk,
                             (jnp.int32(0), jnp.int32(0)))

    pad_base = wid * RBATCH
    for j in range(RBATCH // L):
        pos = (cur + j * L + iota) & (RING - 1)
        plsc.store_scatter(mid_v, [pos], pad_base + j * L + iota)
        plsc.store_scatter(mloc_v, [pos], trash)

    def dcond(st):
        c2, f2 = st
        return f2 < c2

    def dbody(st):
        c2, f2 = st
        dump(f2)
        return (c2, f2 + 256)

    lax.while_loop(dcond, dbody, (cur, dmp))
    cnt_v[pl.ds(0, L)] = jnp.full((L,), 0, jnp.int32) + cur
    pltpu.sync_copy(cnt_v, cnt_hbm.at[pl.ds(pl.multiple_of(wid * L, L), L)])


def _scan(dst):
    f = pl.kernel(
        _scan_body,
        out_type=[
            jax.ShapeDtypeStruct((NW * CAP,), jnp.int32),
            jax.ShapeDtypeStruct((NW * CAP,), jnp.int32),
            jax.ShapeDtypeStruct((NW * L,), jnp.int32),
        ],
        mesh=plsc.VectorSubcoreMesh(core_axis_name="c", subcore_axis_name="s"),
        compiler_params=pltpu.CompilerParams(needs_layout_passes=False),
        scratch_types=[
            pltpu.VMEM((SCHUNK,), jnp.int32),
            pltpu.VMEM((RING + L,), jnp.int32),
            pltpu.VMEM((RING + L,), jnp.int32),
            pltpu.VMEM((L,), jnp.int32),
        ],
    )
    return f(dst)


def _flush_body(h2_hbm, loc_hbm, eid_hbm, cnt_hbm, agg_hbm,
                lbuf_v, ebuf_v, cnt_v, rows_v, agg_v, agg2_v, sem):
    wid = lax.axis_index("s") * NC + lax.axis_index("c")
    lo = wid * RPT
    neg = jnp.full((L,), _NEG, dtype=jnp.float32)

    def init(i, carry):
        for cc in range(D // L):
            agg_v[i, pl.ds(cc * L, L)] = neg
            agg2_v[i, pl.ds(cc * L, L)] = neg
        return carry

    lax.fori_loop(0, RPT + 1, init, 0)

    pltpu.sync_copy(cnt_hbm.at[pl.ds(pl.multiple_of(wid * L, L), L)], cnt_v)
    cnt = cnt_v[pl.ds(0, L)][0]
    nb = (cnt + RBATCH - 1) // RBATCH

    def batch(b, carry):
        boff = pl.multiple_of(b * RBATCH, RBATCH)
        hoff = pl.multiple_of(wid * CAP + boff, RBATCH)
        pltpu.sync_copy(loc_hbm.at[pl.ds(hoff, RBATCH)], lbuf_v.at[pl.ds(0, RBATCH)])
        pltpu.sync_copy(eid_hbm.at[pl.ds(hoff, RBATCH)], ebuf_v)
        d1 = pltpu.async_copy(h2_hbm.at[ebuf_v.at[pl.ds(0, 128)]],
                              rows_v.at[pl.ds(0, 128)], sem)
        d2 = pltpu.async_copy(h2_hbm.at[ebuf_v.at[pl.ds(128, 128)]],
                              rows_v.at[pl.ds(128, 128)], sem)
        d1.wait()
        d2.wait()

        def rmw(i, c2):
            r1 = lbuf_v[pl.ds(i, L)][0]
            r2 = lbuf_v[pl.ds(i + 128, L)][0]
            for cc in range(D // L):
                sl = pl.ds(cc * L, L)
                agg_v[r1, sl] = jnp.maximum(agg_v[r1, sl], rows_v[i, sl])
                agg2_v[r2, sl] = jnp.maximum(agg2_v[r2, sl], rows_v[i + 128, sl])
            return c2

        lax.fori_loop(0, RBATCH // 2, rmw, 0)
        return carry

    lax.fori_loop(0, nb, batch, 0)

    def merge(i, carry):
        for cc in range(D // L):
            sl = pl.ds(cc * L, L)
            agg_v[i, sl] = jnp.maximum(agg_v[i, sl], agg2_v[i, sl])
        return carry

    lax.fori_loop(0, RPT, merge, 0)
    pltpu.sync_copy(agg_v.at[pl.ds(0, RPT)], agg_hbm.at[pl.ds(lo, RPT)])


def _flush(H2, LOC, EID, CNT):
    f = pl.kernel(
        _flush_body,
        out_type=jax.ShapeDtypeStruct((NPAD, D), jnp.float32),
        mesh=plsc.VectorSubcoreMesh(core_axis_name="c", subcore_axis_name="s"),
        compiler_params=pltpu.CompilerParams(needs_layout_passes=False),
        scratch_types=[
            pltpu.VMEM((RBATCH + L,), jnp.int32),
            pltpu.VMEM((RBATCH,), jnp.int32),
            pltpu.VMEM((L,), jnp.int32),
            pltpu.VMEM((RBATCH, D), jnp.float32),
            pltpu.VMEM((RPT + 1, D), jnp.float32),
            pltpu.VMEM((RPT + 1, D), jnp.float32),
            pltpu.SemaphoreType.DMA,
        ],
    )
    return f(H2, LOC, EID, CNT)


def _bn_body(agg_ref, gamma_ref, beta_ref, y_ref):
    a = agg_ref[...]
    a = jnp.where(a == _NEG, 0.0, a)
    mean = jnp.mean(a, axis=0, keepdims=True)
    var = jnp.mean((a - mean) ** 2, axis=0, keepdims=True)
    y_ref[...] = gamma_ref[...] * (a - mean) / jnp.sqrt(var + 1e-5) + beta_ref[...]


def _bn(agg, gamma, beta):
    return pl.pallas_call(
        _bn_body,
        in_specs=[
            pl.BlockSpec((N, D), lambda: (0, 0)),
            pl.BlockSpec((1, D), lambda: (0, 0)),
            pl.BlockSpec((1, D), lambda: (0, 0)),
        ],
        out_specs=pl.BlockSpec((N, D), lambda: (0, 0)),
        out_shape=jax.ShapeDtypeStruct((N, D), jnp.float32),
    )(agg, gamma, beta)


def kernel(x, edge_index, edge_attr, W1, b1, W2, b2, gamma, beta):
    src = edge_index[0]
    dst = edge_index[1]
    A, B = _pre(x, W1, b1.reshape(1, D))
    G1, G2 = _gather(A, B, dst, src)
    H2 = _mlp(G1, G2, W2, b2.reshape(1, D))
    LOC, EID, CNT = _scan(dst)
    aggp = _flush(H2, LOC, EID, CNT)
    y = _bn(aggp[:N], gamma.reshape(1, D), beta.reshape(1, D))
    return (y, edge_index, edge_attr)


# speedup vs baseline: 1.2370x; 1.0107x over previous
import functools

import jax
import jax.numpy as jnp
from jax import lax
from jax.experimental import pallas as pl
from jax.experimental.pallas import tpu as pltpu
from jax.experimental.pallas import tpu_sc as plsc

N = 10000
E = 320000
D = 128
NC, NS, L = 2, 16, 16
NW = NC * NS
EPW = E // NW
RPT = 320
NPAD = NW * RPT
GCHUNK = 200
GSUB = 40
SCHUNK = 3200
RING = 4096
RBATCH = 256
CAP = E + RBATCH

_NEG = float("-inf")


def _pre_body(x_ref, w1_ref, b1_ref, a_ref, b_ref):
    xb = x_ref[...]
    w1a = w1_ref[:D, :]
    w1b = w1_ref[D:, :]
    a_ref[...] = jnp.dot(xb, w1a - w1b, preferred_element_type=jnp.float32) + b1_ref[...]
    b_ref[...] = jnp.dot(xb, w1b, preferred_element_type=jnp.float32)


def _pre(x, W1, b1):
    grid = 10
    blk = N // grid
    return pl.pallas_call(
        _pre_body,
        grid=(grid,),
        in_specs=[
            pl.BlockSpec((blk, D), lambda i: (i, 0)),
            pl.BlockSpec((2 * D, D), lambda i: (0, 0)),
            pl.BlockSpec((1, D), lambda i: (0, 0)),
        ],
        out_specs=[
            pl.BlockSpec((blk, D), lambda i: (i, 0)),
            pl.BlockSpec((blk, D), lambda i: (i, 0)),
        ],
        out_shape=[
            jax.ShapeDtypeStruct((N, D), jnp.float32),
            jax.ShapeDtypeStruct((N, D), jnp.float32),
        ],
    )(x, W1, b1)


def _gather_body(a_hbm, b_hbm, dst_hbm, src_hbm, g1_hbm, g2_hbm,
                 idxd0_v, idxs0_v, idxd1_v, idxs1_v,
                 bufa0_v, bufb0_v, bufa1_v, bufb1_v, sem0, sem1):
    wid = lax.axis_index("s") * NC + lax.axis_index("c")
    ebase = wid * EPW
    idxd = (idxd0_v, idxd1_v)
    idxs = (idxs0_v, idxs1_v)
    bufa = (bufa0_v, bufb0_v)
    bufb = (bufa1_v, bufb1_v)
    bufs_a = (bufa0_v, bufa1_v)
    bufs_b = (bufb0_v, bufb1_v)
    sems = (sem0, sem1)

    def fire(c, par):
        cbase = ebase + c * GCHUNK
        pltpu.sync_copy(dst_hbm.at[pl.ds(cbase, GCHUNK)], idxd[par])
        pltpu.sync_copy(src_hbm.at[pl.ds(cbase, GCHUNK)], idxs[par])
        for k in range(GCHUNK // GSUB):
            sl = pl.ds(k * GSUB, GSUB)
            pltpu.async_copy(a_hbm.at[idxd[par].at[sl]], bufs_a[par].at[sl], sems[par])
            pltpu.async_copy(b_hbm.at[idxs[par].at[sl]], bufs_b[par].at[sl], sems[par])

    def wait_write(c, par):
        cbase = ebase + c * GCHUNK
        pltpu.make_async_copy(a_hbm.at[pl.ds(0, GCHUNK)], bufs_a[par], sems[par]).wait()
        pltpu.make_async_copy(b_hbm.at[pl.ds(0, GCHUNK)], bufs_b[par], sems[par]).wait()
        pltpu.sync_copy(bufs_a[par], g1_hbm.at[pl.ds(cbase, GCHUNK)])
        pltpu.sync_copy(bufs_b[par], g2_hbm.at[pl.ds(cbase, GCHUNK)])

    nch = EPW // GCHUNK
    fire(0, 0)

    def body(i, carry):
        c0 = i * 2
        fire(c0 + 1, 1)
        wait_write(c0, 0)
        fire(c0 + 2, 0)
        wait_write(c0 + 1, 1)
        return carry

    lax.fori_loop(0, nch // 2 - 1, body, 0)
    fire(nch - 1, 1)
    wait_write(nch - 2, 0)
    wait_write(nch - 1, 1)


def _gather(A, B, dst, src):
    f = pl.kernel(
        _gather_body,
        out_type=[
            jax.ShapeDtypeStruct((E, D), jnp.float32),
            jax.ShapeDtypeStruct((E, D), jnp.float32),
        ],
        mesh=plsc.VectorSubcoreMesh(core_axis_name="c", subcore_axis_name="s"),
        compiler_params=pltpu.CompilerParams(needs_layout_passes=False),
        scratch_types=[
            pltpu.VMEM((GCHUNK,), jnp.int32),
            pltpu.VMEM((GCHUNK,), jnp.int32),
            pltpu.VMEM((GCHUNK,), jnp.int32),
            pltpu.VMEM((GCHUNK,), jnp.int32),
            pltpu.VMEM((GCHUNK, D), jnp.float32),
            pltpu.VMEM((GCHUNK, D), jnp.float32),
            pltpu.VMEM((GCHUNK, D), jnp.float32),
            pltpu.VMEM((GCHUNK, D), jnp.float32),
            pltpu.SemaphoreType.DMA,
            pltpu.SemaphoreType.DMA,
        ],
    )
    return f(A, B, dst, src)


def _mlp_body(g1_ref, g2_ref, w2_ref, b2_ref, h2_ref):
    h1 = g1_ref[...] + g2_ref[...]
    m = h1 * jnp.tanh(jax.nn.softplus(h1))
    h2_ref[...] = jnp.dot(m, w2_ref[...], preferred_element_type=jnp.float32) + b2_ref[...]


def _mlp(G1, G2, W2, b2):
    blk = 512
    grid = E // blk
    return pl.pallas_call(
        _mlp_body,
        grid=(grid,),
        in_specs=[
            pl.BlockSpec((blk, D), lambda i: (i, 0)),
            pl.BlockSpec((blk, D), lambda i: (i, 0)),
            pl.BlockSpec((D, D), lambda i: (0, 0)),
            pl.BlockSpec((1, D), lambda i: (0, 0)),
        ],
        out_specs=pl.BlockSpec((blk, D), lambda i: (i, 0)),
        out_shape=jax.ShapeDtypeStruct((E, D), jnp.float32),
    )(G1, G2, W2, b2)


def _scan_body(dst_hbm, loc_hbm, eid_hbm, cnt_hbm, dstb_v, mloc_v, mid_v, cnt_v):
    wid = lax.axis_index("s") * NC + lax.axis_index("c")
    lo = wid * RPT
    hi = lo + RPT
    iota = lax.iota(jnp.int32, L)
    trash = jnp.full((L,), RPT, jnp.int32)

    def dump(dmp):
        roff = pl.multiple_of(dmp & (RING - 1), 256)
        hoff = pl.multiple_of(wid * CAP + dmp, 256)
        pltpu.sync_copy(mloc_v.at[pl.ds(roff, 256)], loc_hbm.at[pl.ds(hoff, 256)])
        pltpu.sync_copy(mid_v.at[pl.ds(roff, 256)], eid_hbm.at[pl.ds(hoff, 256)])

    def chunk(ci, carry):
        cur0, dmp0 = carry
        cbase = ci * SCHUNK
        pltpu.sync_copy(dst_hbm.at[pl.ds(cbase, SCHUNK)], dstb_v)

        def scan32(g, cur):
            d16a = dstb_v[pl.ds(g * 2 * L, L)]
            d16b = dstb_v[pl.ds(g * 2 * L + L, L)]
            ma = (d16a >= lo) & (d16a < hi)
            mb = (d16b >= lo) & (d16b < hi)
            pca = plsc.cumsum(jnp.where(ma, jnp.int32(1), jnp.int32(0)))
            pcb = plsc.cumsum(jnp.where(mb, jnp.int32(1), jnp.int32(0)))
            ca = pca[15]
            posa = jnp.where(ma, (cur + pca - 1) & (RING - 1), RING + iota)
            posb = jnp.where(mb, (cur + ca + pcb - 1) & (RING - 1), RING + iota)
            plsc.store_scatter(mloc_v, [posa], d16a - lo)
            plsc.store_scatter(mid_v, [posa], cbase + g * 2 * L + iota)
            plsc.store_scatter(mloc_v, [posb], d16b - lo)
            plsc.store_scatter(mid_v, [posb], cbase + g * 2 * L + L + iota)
            return cur + ca + pcb[15]

        cur1 = lax.fori_loop(0, SCHUNK // (2 * L), scan32, cur0)

        def wcond(st):
            c2, f2 = st
            return c2 - f2 >= 256

        def wbody(st):
            c2, f2 = st
            dump(f2)
            return (c2, f2 + 256)

        return lax.while_loop(wcond, wbody, (cur1, dmp0))

    cur, dmp = lax.fori_loop(0, E // SCHUNK, chunk,
                             (jnp.int32(0), jnp.int32(0)))

    pad_base = wid * RBATCH
    for j in range(RBATCH // L):
        pos = (cur + j * L + iota) & (RING - 1)
        plsc.store_scatter(mid_v, [pos], pad_base + j * L + iota)
        plsc.store_scatter(mloc_v, [pos], trash)

    def dcond(st):
        c2, f2 = st
        return f2 < c2

    def dbody(st):
        c2, f2 = st
        dump(f2)
        return (c2, f2 + 256)

    lax.while_loop(dcond, dbody, (cur, dmp))
    cnt_v[pl.ds(0, L)] = jnp.full((L,), 0, jnp.int32) + cur
    pltpu.sync_copy(cnt_v, cnt_hbm.at[pl.ds(pl.multiple_of(wid * L, L), L)])


def _scan(dst):
    f = pl.kernel(
        _scan_body,
        out_type=[
            jax.ShapeDtypeStruct((NW * CAP,), jnp.int32),
            jax.ShapeDtypeStruct((NW * CAP,), jnp.int32),
            jax.ShapeDtypeStruct((NW * L,), jnp.int32),
        ],
        mesh=plsc.VectorSubcoreMesh(core_axis_name="c", subcore_axis_name="s"),
        compiler_params=pltpu.CompilerParams(needs_layout_passes=False),
        scratch_types=[
            pltpu.VMEM((SCHUNK,), jnp.int32),
            pltpu.VMEM((RING + L,), jnp.int32),
            pltpu.VMEM((RING + L,), jnp.int32),
            pltpu.VMEM((L,), jnp.int32),
        ],
    )
    return f(dst)


def _flush_body(h2_hbm, loc_hbm, eid_hbm, cnt_hbm, agg_hbm,
                lbuf_v, ebuf_v, cnt_v, rows_v, agg_v, agg2_v, sem):
    wid = lax.axis_index("s") * NC + lax.axis_index("c")
    lo = wid * RPT
    neg = jnp.full((L,), _NEG, dtype=jnp.float32)

    def init(i, carry):
        for cc in range(D // L):
            agg_v[i, pl.ds(cc * L, L)] = neg
            agg2_v[i, pl.ds(cc * L, L)] = neg
        return carry

    lax.fori_loop(0, RPT + 1, init, 0)

    pltpu.sync_copy(cnt_hbm.at[pl.ds(pl.multiple_of(wid * L, L), L)], cnt_v)
    cnt = cnt_v[pl.ds(0, L)][0]
    nb = (cnt + RBATCH - 1) // RBATCH

    def batch(b, carry):
        boff = pl.multiple_of(b * RBATCH, RBATCH)
        hoff = pl.multiple_of(wid * CAP + boff, RBATCH)
        pltpu.sync_copy(loc_hbm.at[pl.ds(hoff, RBATCH)], lbuf_v.at[pl.ds(0, RBATCH)])
        pltpu.sync_copy(eid_hbm.at[pl.ds(hoff, RBATCH)], ebuf_v)
        d1 = pltpu.async_copy(h2_hbm.at[ebuf_v.at[pl.ds(0, 128)]],
                              rows_v.at[pl.ds(0, 128)], sem)
        d2 = pltpu.async_copy(h2_hbm.at[ebuf_v.at[pl.ds(128, 128)]],
                              rows_v.at[pl.ds(128, 128)], sem)
        d1.wait()
        d2.wait()

        def rmw(i, c2):
            r1 = lbuf_v[pl.ds(i, L)][0]
            r2 = lbuf_v[pl.ds(i + 128, L)][0]
            for cc in range(D // L):
                sl = pl.ds(cc * L, L)
                agg_v[r1, sl] = jnp.maximum(agg_v[r1, sl], rows_v[i, sl])
                agg2_v[r2, sl] = jnp.maximum(agg2_v[r2, sl], rows_v[i + 128, sl])
            return c2

        lax.fori_loop(0, RBATCH // 2, rmw, 0)
        return carry

    lax.fori_loop(0, nb, batch, 0)

    def merge(i, carry):
        for cc in range(D // L):
            sl = pl.ds(cc * L, L)
            agg_v[i, sl] = jnp.maximum(agg_v[i, sl], agg2_v[i, sl])
        return carry

    lax.fori_loop(0, RPT, merge, 0)
    pltpu.sync_copy(agg_v.at[pl.ds(0, RPT)], agg_hbm.at[pl.ds(lo, RPT)])


def _flush(H2, LOC, EID, CNT):
    f = pl.kernel(
        _flush_body,
        out_type=jax.ShapeDtypeStruct((NPAD, D), jnp.float32),
        mesh=plsc.VectorSubcoreMesh(core_axis_name="c", subcore_axis_name="s"),
        compiler_params=pltpu.CompilerParams(needs_layout_passes=False),
        scratch_types=[
            pltpu.VMEM((RBATCH + L,), jnp.int32),
            pltpu.VMEM((RBATCH,), jnp.int32),
            pltpu.VMEM((L,), jnp.int32),
            pltpu.VMEM((RBATCH, D), jnp.float32),
            pltpu.VMEM((RPT + 1, D), jnp.float32),
            pltpu.VMEM((RPT + 1, D), jnp.float32),
            pltpu.SemaphoreType.DMA,
        ],
    )
    return f(H2, LOC, EID, CNT)


def _bn_body(agg_ref, gamma_ref, beta_ref, y_ref):
    a = agg_ref[...]
    a = jnp.where(a == _NEG, 0.0, a)
    mean = jnp.mean(a, axis=0, keepdims=True)
    var = jnp.mean((a - mean) ** 2, axis=0, keepdims=True)
    y_ref[...] = gamma_ref[...] * (a - mean) / jnp.sqrt(var + 1e-5) + beta_ref[...]


def _bn(agg, gamma, beta):
    return pl.pallas_call(
        _bn_body,
        in_specs=[
            pl.BlockSpec((N, D), lambda: (0, 0)),
            pl.BlockSpec((1, D), lambda: (0, 0)),
            pl.BlockSpec((1, D), lambda: (0, 0)),
        ],
        out_specs=pl.BlockSpec((N, D), lambda: (0, 0)),
        out_shape=jax.ShapeDtypeStruct((N, D), jnp.float32),
    )(agg, gamma, beta)


def kernel(x, edge_index, edge_attr, W1, b1, W2, b2, gamma, beta):
    src = edge_index[0]
    dst = edge_index[1]
    A, B = _pre(x, W1, b1.reshape(1, D))
    G1, G2 = _gather(A, B, dst, src)
    H2 = _mlp(G1, G2, W2, b2.reshape(1, D))
    LOC, EID, CNT = _scan(dst)
    aggp = _flush(H2, LOC, EID, CNT)
    y = _bn(aggp[:N], gamma.reshape(1, D), beta.reshape(1, D))
    return (y, edge_index, edge_attr)


# speedup vs baseline: 1.2641x; 1.0219x over previous
import functools

import jax
import jax.numpy as jnp
from jax import lax
from jax.experimental import pallas as pl
from jax.experimental.pallas import tpu as pltpu
from jax.experimental.pallas import tpu_sc as plsc

N = 10000
E = 320000
D = 128
NC, NS, L = 2, 16, 16
NW = NC * NS
EPW = E // NW
RPT = 320
NPAD = NW * RPT
GCHUNK = 200
GSUB = 40
SCHUNK = 3200
RING = 4096
RBATCH = 256
CAP = E + 1024

_NEG = float("-inf")


def _pre_body(x_ref, w1_ref, b1_ref, a_ref, b_ref):
    xb = x_ref[...]
    w1a = w1_ref[:D, :]
    w1b = w1_ref[D:, :]
    a_ref[...] = jnp.dot(xb, w1a - w1b, preferred_element_type=jnp.float32) + b1_ref[...]
    b_ref[...] = jnp.dot(xb, w1b, preferred_element_type=jnp.float32)


def _pre(x, W1, b1):
    grid = 10
    blk = N // grid
    return pl.pallas_call(
        _pre_body,
        grid=(grid,),
        in_specs=[
            pl.BlockSpec((blk, D), lambda i: (i, 0)),
            pl.BlockSpec((2 * D, D), lambda i: (0, 0)),
            pl.BlockSpec((1, D), lambda i: (0, 0)),
        ],
        out_specs=[
            pl.BlockSpec((blk, D), lambda i: (i, 0)),
            pl.BlockSpec((blk, D), lambda i: (i, 0)),
        ],
        out_shape=[
            jax.ShapeDtypeStruct((N, D), jnp.float32),
            jax.ShapeDtypeStruct((N, D), jnp.float32),
        ],
    )(x, W1, b1)


def _gather_body(a_hbm, b_hbm, dst_hbm, src_hbm, g1_hbm, g2_hbm,
                 idxd0_v, idxs0_v, idxd1_v, idxs1_v,
                 bufa0_v, bufb0_v, bufa1_v, bufb1_v, sem0, sem1):
    wid = lax.axis_index("s") * NC + lax.axis_index("c")
    ebase = wid * EPW
    idxd = (idxd0_v, idxd1_v)
    idxs = (idxs0_v, idxs1_v)
    bufa = (bufa0_v, bufb0_v)
    bufb = (bufa1_v, bufb1_v)
    bufs_a = (bufa0_v, bufa1_v)
    bufs_b = (bufb0_v, bufb1_v)
    sems = (sem0, sem1)

    def fire(c, par):
        cbase = ebase + c * GCHUNK
        pltpu.sync_copy(dst_hbm.at[pl.ds(cbase, GCHUNK)], idxd[par])
        pltpu.sync_copy(src_hbm.at[pl.ds(cbase, GCHUNK)], idxs[par])
        for k in range(GCHUNK // GSUB):
            sl = pl.ds(k * GSUB, GSUB)
            pltpu.async_copy(a_hbm.at[idxd[par].at[sl]], bufs_a[par].at[sl], sems[par])
            pltpu.async_copy(b_hbm.at[idxs[par].at[sl]], bufs_b[par].at[sl], sems[par])

    def wait_write(c, par):
        cbase = ebase + c * GCHUNK
        pltpu.make_async_copy(a_hbm.at[pl.ds(0, GCHUNK)], bufs_a[par], sems[par]).wait()
        pltpu.make_async_copy(b_hbm.at[pl.ds(0, GCHUNK)], bufs_b[par], sems[par]).wait()
        pltpu.sync_copy(bufs_a[par], g1_hbm.at[pl.ds(cbase, GCHUNK)])
        pltpu.sync_copy(bufs_b[par], g2_hbm.at[pl.ds(cbase, GCHUNK)])

    nch = EPW // GCHUNK
    fire(0, 0)

    def body(i, carry):
        c0 = i * 2
        fire(c0 + 1, 1)
        wait_write(c0, 0)
        fire(c0 + 2, 0)
        wait_write(c0 + 1, 1)
        return carry

    lax.fori_loop(0, nch // 2 - 1, body, 0)
    fire(nch - 1, 1)
    wait_write(nch - 2, 0)
    wait_write(nch - 1, 1)


def _gather(A, B, dst, src):
    f = pl.kernel(
        _gather_body,
        out_type=[
            jax.ShapeDtypeStruct((E, D), jnp.float32),
            jax.ShapeDtypeStruct((E, D), jnp.float32),
        ],
        mesh=plsc.VectorSubcoreMesh(core_axis_name="c", subcore_axis_name="s"),
        compiler_params=pltpu.CompilerParams(needs_layout_passes=False),
        scratch_types=[
            pltpu.VMEM((GCHUNK,), jnp.int32),
            pltpu.VMEM((GCHUNK,), jnp.int32),
            pltpu.VMEM((GCHUNK,), jnp.int32),
            pltpu.VMEM((GCHUNK,), jnp.int32),
            pltpu.VMEM((GCHUNK, D), jnp.float32),
            pltpu.VMEM((GCHUNK, D), jnp.float32),
            pltpu.VMEM((GCHUNK, D), jnp.float32),
            pltpu.VMEM((GCHUNK, D), jnp.float32),
            pltpu.SemaphoreType.DMA,
            pltpu.SemaphoreType.DMA,
        ],
    )
    return f(A, B, dst, src)


def _mlp_body(g1_ref, g2_ref, w2_ref, b2_ref, h2_ref):
    h1 = g1_ref[...] + g2_ref[...]
    m = h1 * jnp.tanh(jax.nn.softplus(h1))
    h2_ref[...] = jnp.dot(m, w2_ref[...], preferred_element_type=jnp.float32) + b2_ref[...]


def _mlp(G1, G2, W2, b2):
    blk = 512
    grid = E // blk
    return pl.pallas_call(
        _mlp_body,
        grid=(grid,),
        in_specs=[
            pl.BlockSpec((blk, D), lambda i: (i, 0)),
            pl.BlockSpec((blk, D), lambda i: (i, 0)),
            pl.BlockSpec((D, D), lambda i: (0, 0)),
            pl.BlockSpec((1, D), lambda i: (0, 0)),
        ],
        out_specs=pl.BlockSpec((blk, D), lambda i: (i, 0)),
        out_shape=jax.ShapeDtypeStruct((E, D), jnp.float32),
    )(G1, G2, W2, b2)


def _scan_body(dst_hbm, loc_hbm, eid_hbm, cnt_hbm, dstb_v, mloc_v, mid_v, cnt_v):
    wid = lax.axis_index("s") * NC + lax.axis_index("c")
    lo = wid * RPT
    hi = lo + RPT
    iota = lax.iota(jnp.int32, L)
    trash = jnp.full((L,), RPT, jnp.int32)

    def dump(dmp):
        roff = pl.multiple_of(dmp & (RING - 1), 256)
        hoff = pl.multiple_of(wid * CAP + dmp, 256)
        pltpu.sync_copy(mloc_v.at[pl.ds(roff, 256)], loc_hbm.at[pl.ds(hoff, 256)])
        pltpu.sync_copy(mid_v.at[pl.ds(roff, 256)], eid_hbm.at[pl.ds(hoff, 256)])

    def chunk(ci, carry):
        cur0, dmp0 = carry
        cbase = ci * SCHUNK
        pltpu.sync_copy(dst_hbm.at[pl.ds(cbase, SCHUNK)], dstb_v)

        def scan32(g, cur):
            d16a = dstb_v[pl.ds(g * 2 * L, L)]
            d16b = dstb_v[pl.ds(g * 2 * L + L, L)]
            ma = (d16a >= lo) & (d16a < hi)
            mb = (d16b >= lo) & (d16b < hi)
            pca = plsc.cumsum(jnp.where(ma, jnp.int32(1), jnp.int32(0)))
            pcb = plsc.cumsum(jnp.where(mb, jnp.int32(1), jnp.int32(0)))
            ca = pca[15]
            posa = jnp.where(ma, (cur + pca - 1) & (RING - 1), RING + iota)
            posb = jnp.where(mb, (cur + ca + pcb - 1) & (RING - 1), RING + iota)
            plsc.store_scatter(mloc_v, [posa], d16a - lo)
            plsc.store_scatter(mid_v, [posa], cbase + g * 2 * L + iota)
            plsc.store_scatter(mloc_v, [posb], d16b - lo)
            plsc.store_scatter(mid_v, [posb], cbase + g * 2 * L + L + iota)
            return cur + ca + pcb[15]

        cur1 = lax.fori_loop(0, SCHUNK // (2 * L), scan32, cur0)

        def wcond(st):
            c2, f2 = st
            return c2 - f2 >= 256

        def wbody(st):
            c2, f2 = st
            dump(f2)
            return (c2, f2 + 256)

        return lax.while_loop(wcond, wbody, (cur1, dmp0))

    cur, dmp = lax.fori_loop(0, E // SCHUNK, chunk,
                             (jnp.int32(0), jnp.int32(0)))

    pad_base = wid * 1024

    def pad(j, carry):
        pos = (cur + j * L + iota) & (RING - 1)
        plsc.store_scatter(mid_v, [pos], pad_base + j * L + iota)
        plsc.store_scatter(mloc_v, [pos], trash)
        return carry

    lax.fori_loop(0, 1024 // L, pad, 0)

    def dcond(st):
        c2, f2 = st
        return f2 < c2 + 768

    def dbody(st):
        c2, f2 = st
        dump(f2)
        return (c2, f2 + 256)

    lax.while_loop(dcond, dbody, (cur, dmp))
    cnt_v[pl.ds(0, L)] = jnp.full((L,), 0, jnp.int32) + cur
    pltpu.sync_copy(cnt_v, cnt_hbm.at[pl.ds(pl.multiple_of(wid * L, L), L)])


def _scan(dst):
    f = pl.kernel(
        _scan_body,
        out_type=[
            jax.ShapeDtypeStruct((NW * CAP,), jnp.int32),
            jax.ShapeDtypeStruct((NW * CAP,), jnp.int32),
            jax.ShapeDtypeStruct((NW * L,), jnp.int32),
        ],
        mesh=plsc.VectorSubcoreMesh(core_axis_name="c", subcore_axis_name="s"),
        compiler_params=pltpu.CompilerParams(needs_layout_passes=False),
        scratch_types=[
            pltpu.VMEM((SCHUNK,), jnp.int32),
            pltpu.VMEM((RING + L,), jnp.int32),
            pltpu.VMEM((RING + L,), jnp.int32),
            pltpu.VMEM((L,), jnp.int32),
        ],
    )
    return f(dst)


def _flush_body(h2_hbm, loc_hbm, eid_hbm, cnt_hbm, agg_hbm,
                lbuf0_v, lbuf1_v, ebuf0_v, ebuf1_v, cnt_v,
                rows0_v, rows1_v, agg_v, sem0, sem1):
    wid = lax.axis_index("s") * NC + lax.axis_index("c")
    lo = wid * RPT
    neg = jnp.full((L,), _NEG, dtype=jnp.float32)
    lbuf = (lbuf0_v, lbuf1_v)
    ebuf = (ebuf0_v, ebuf1_v)
    rows = (rows0_v, rows1_v)
    sems = (sem0, sem1)

    def init(i, carry):
        for cc in range(D // L):
            agg_v[i, pl.ds(cc * L, L)] = neg
        return carry

    lax.fori_loop(0, RPT + 1, init, 0)

    pltpu.sync_copy(cnt_hbm.at[pl.ds(pl.multiple_of(wid * L, L), L)], cnt_v)
    cnt = cnt_v[pl.ds(0, L)][0]
    nb2 = ((cnt + 511) // 512) * 2

    def fire(c, par):
        hoff = pl.multiple_of(wid * CAP + c * RBATCH, RBATCH)
        pltpu.sync_copy(loc_hbm.at[pl.ds(hoff, RBATCH)], lbuf[par].at[pl.ds(0, RBATCH)])
        pltpu.sync_copy(eid_hbm.at[pl.ds(hoff, RBATCH)], ebuf[par])
        pltpu.async_copy(h2_hbm.at[ebuf[par].at[pl.ds(0, 128)]],
                         rows[par].at[pl.ds(0, 128)], sems[par])
        pltpu.async_copy(h2_hbm.at[ebuf[par].at[pl.ds(128, 128)]],
                         rows[par].at[pl.ds(128, 128)], sems[par])

    def waitb(par):
        pltpu.make_async_copy(h2_hbm.at[pl.ds(0, RBATCH)], rows[par], sems[par]).wait()

    def rmw_batch(par):
        def rmw(i, c2):
            r = lbuf[par][pl.ds(i, L)][0]
            for cc in range(D // L):
                sl = pl.ds(cc * L, L)
                agg_v[r, sl] = jnp.maximum(agg_v[r, sl], rows[par][i, sl])
            return c2

        lax.fori_loop(0, RBATCH, rmw, 0)

    fire(0, 0)
    fire(1, 1)

    def body(j, carry):
        waitb(0)
        rmw_batch(0)
        fire(2 * j + 2, 0)
        waitb(1)
        rmw_batch(1)
        fire(2 * j + 3, 1)
        return carry

    lax.fori_loop(0, nb2 // 2, body, 0)
    waitb(0)
    waitb(1)
    pltpu.sync_copy(agg_v.at[pl.ds(0, RPT)], agg_hbm.at[pl.ds(lo, RPT)])


def _flush(H2, LOC, EID, CNT):
    f = pl.kernel(
        _flush_body,
        out_type=jax.ShapeDtypeStruct((NPAD, D), jnp.float32),
        mesh=plsc.VectorSubcoreMesh(core_axis_name="c", subcore_axis_name="s"),
        compiler_params=pltpu.CompilerParams(needs_layout_passes=False),
        scratch_types=[
            pltpu.VMEM((RBATCH + L,), jnp.int32),
            pltpu.VMEM((RBATCH + L,), jnp.int32),
            pltpu.VMEM((RBATCH,), jnp.int32),
            pltpu.VMEM((RBATCH,), jnp.int32),
            pltpu.VMEM((L,), jnp.int32),
            pltpu.VMEM((RBATCH, D), jnp.float32),
            pltpu.VMEM((RBATCH, D), jnp.float32),
            pltpu.VMEM((RPT + 1, D), jnp.float32),
            pltpu.SemaphoreType.DMA,
            pltpu.SemaphoreType.DMA,
        ],
    )
    return f(H2, LOC, EID, CNT)


def _bn_body(agg_ref, gamma_ref, beta_ref, y_ref):
    a = agg_ref[...]
    a = jnp.where(a == _NEG, 0.0, a)
    mean = jnp.mean(a, axis=0, keepdims=True)
    var = jnp.mean((a - mean) ** 2, axis=0, keepdims=True)
    y_ref[...] = gamma_ref[...] * (a - mean) / jnp.sqrt(var + 1e-5) + beta_ref[...]


def _bn(agg, gamma, beta):
    return pl.pallas_call(
        _bn_body,
        in_specs=[
            pl.BlockSpec((N, D), lambda: (0, 0)),
            pl.BlockSpec((1, D), lambda: (0, 0)),
            pl.BlockSpec((1, D), lambda: (0, 0)),
        ],
        out_specs=pl.BlockSpec((N, D), lambda: (0, 0)),
        out_shape=jax.ShapeDtypeStruct((N, D), jnp.float32),
    )(agg, gamma, beta)


def kernel(x, edge_index, edge_attr, W1, b1, W2, b2, gamma, beta):
    src = edge_index[0]
    dst = edge_index[1]
    A, B = _pre(x, W1, b1.reshape(1, D))
    G1, G2 = _gather(A, B, dst, src)
    H2 = _mlp(G1, G2, W2, b2.reshape(1, D))
    LOC, EID, CNT = _scan(dst)
    aggp = _flush(H2, LOC, EID, CNT)
    y = _bn(aggp[:N], gamma.reshape(1, D), beta.reshape(1, D))
    return (y, edge_index, edge_attr)
